# baseline (speedup 1.0000x reference)
import jax
import jax.numpy as jnp
from jax import lax
from jax.experimental import pallas as pl
from jax.experimental.pallas import tpu as pltpu

B = 16384
V = 1000
K = 11468

_CH = 1024
_NCH = B // _CH
_NBUF = 6


def _ce_body(x_hbm, t_ref, s_ref, g_ref, bufs, sems):
    def start(c):
        b = lax.rem(c, _NBUF)
        pltpu.make_async_copy(
            x_hbm.at[pl.ds(c * _CH, _CH), :], bufs.at[b], sems.at[b]).start()

    for c in range(_NBUF):
        start(c)

    col = lax.broadcasted_iota(jnp.int32, (_CH, V), 1)

    def step(c, _):
        b = lax.rem(c, _NBUF)
        pltpu.make_async_copy(
            x_hbm.at[pl.ds(c * _CH, _CH), :], bufs.at[b], sems.at[b]).wait()
        x = bufs[b]
        tc = jnp.transpose(t_ref[pl.ds(c, 1), :])
        s = jnp.sum(jnp.exp(jnp.minimum(x, 80.0)), axis=1, keepdims=True)
        g = jnp.sum(jnp.where(col == tc, x, 0.0), axis=1, keepdims=True)
        s_ref[pl.ds(c, 1), :] = jnp.transpose(s)
        g_ref[pl.ds(c, 1), :] = jnp.transpose(g)

        @pl.when(c + _NBUF < _NCH)
        def _():
            start(c + _NBUF)
        return 0

    lax.fori_loop(0, _NCH, step, 0)


def _ce_pass(logit, t2d):
    return pl.pallas_call(
        _ce_body,
        in_specs=[pl.BlockSpec(memory_space=pl.ANY),
                  pl.BlockSpec(memory_space=pltpu.MemorySpace.VMEM)],
        out_specs=[pl.BlockSpec(memory_space=pltpu.MemorySpace.VMEM)] * 2,
        out_shape=[jax.ShapeDtypeStruct((_NCH, _CH), jnp.float32)] * 2,
        scratch_shapes=[pltpu.VMEM((_NBUF, _CH, V), jnp.float32),
                        pltpu.SemaphoreType.DMA((_NBUF,))],
    )(logit, t2d)


def _topk_body(s_ref, g_ref, o_ref):
    loss = jnp.maximum(jnp.log(s_ref[...]) - g_ref[...], 0.0)
    keys = lax.bitcast_convert_type(loss, jnp.int32)

    def count_ge(thr):
        return jnp.sum((keys >= thr).astype(jnp.int32))

    def body(_, carry):
        lo, hi = carry
        mid = lo + (hi - lo) // 2
        take = count_ge(mid) >= K
        return jnp.where(take, mid, lo), jnp.where(take, hi, mid)

    lo, _ = lax.fori_loop(
        0, 31, body, (jnp.int32(0), jnp.int32(0x7F800001)))
    v = lax.bitcast_convert_type(lo, jnp.float32)
    gt = keys >= lo + 1
    c_gt = jnp.sum(gt.astype(jnp.int32))
    s_gt = jnp.sum(jnp.where(gt, loss, 0.0))
    res = (s_gt + (K - c_gt).astype(jnp.float32) * v) / K
    o_ref[...] = res[None, None]


def _topk_mean(s, g):
    return pl.pallas_call(
        _topk_body,
        in_specs=[pl.BlockSpec((_NCH, _CH), lambda: (0, 0))] * 2,
        out_specs=pl.BlockSpec((1, 1), lambda: (0, 0)),
        out_shape=jax.ShapeDtypeStruct((1, 1), jnp.float32),
    )(s, g)


def kernel(logit, t):
    t2d = t.astype(jnp.int32).reshape(_NCH, _CH)
    s, g = _ce_pass(logit, t2d)
    out = _topk_mean(s, g)
    return out[0, 0]
